# baseline (speedup 1.0000x reference)
import functools

import jax
import jax.numpy as jnp
from jax import lax
from jax.experimental import pallas as pl
from jax.experimental.pallas import tpu as pltpu
from jax.experimental.pallas import tpu_sc as plsc

B, T, K = 64, 2048, 128
NC, NS = 2, 16
NW = NC * NS
ROWS_PER_W = B // NW
C = 128
NCH = T // C
L = 16
KV = K // L


def _sc_body(unaries_hbm, lengths_hbm, out_hbm, len_v, buf0, buf1, out_v,
             sem0, sem1):
    cid = lax.axis_index("c")
    sid = lax.axis_index("s")
    wid = sid * NC + cid
    r0 = wid * ROWS_PER_W

    pltpu.sync_copy(lengths_hbm, len_v)
    iota0 = lax.iota(jnp.int32, L)
    lv = plsc.load_gather(len_v, [r0 + jnp.minimum(iota0, 1)])
    ln0 = jnp.minimum(jnp.maximum(lv[0], 0), T)
    ln1 = jnp.minimum(jnp.maximum(lv[1], 0), T)
    n0 = (ln0 + C - 1) // C
    n1 = (ln1 + C - 1) // C
    ntot = n0 + n1

    iota = lax.iota(jnp.int32, L)
    idx_c = [iota + j * L for j in range(KV)]

    def chunk_src(j):
        in_r1 = (j >= n0).astype(jnp.int32)
        t0 = jnp.where(j < n0, j, j - n0) * C
        return r0 + in_r1, in_r1, t0

    def start(j, buf, sem):
        row, _, t0 = chunk_src(j)
        pltpu.async_copy(unaries_hbm.at[row, pl.ds(t0, C)], buf, sem)

    def wait(buf, sem):
        pltpu.make_async_copy(unaries_hbm.at[0, pl.ds(0, C)], buf, sem).wait()

    def combine(av, ai, bv, bi):
        m = bv > av
        return jnp.where(m, bv, av), jnp.where(m, bi, ai)

    def compute_chunk(j, buf):
        _, rloc, t0 = chunk_src(j)

        def grp(g, _):
            base = g * L
            acc = jnp.zeros((L,), jnp.int32)
            for lane in range(L):
                t = base + lane
                vs = [buf[t, pl.ds(k * L, L)] for k in range(KV)]
                l1 = [combine(vs[2 * k], idx_c[2 * k], vs[2 * k + 1],
                              idx_c[2 * k + 1]) for k in range(4)]
                l2 = [combine(*l1[0], *l1[1]), combine(*l1[2], *l1[3])]
                bv, bi = combine(*l2[0], *l2[1])
                gmax = jnp.max(bv)
                cand = jnp.where(bv == gmax, bi, K)
                acc = jnp.where(iota == lane, jnp.min(cand), acc)
            out_v[rloc, pl.ds(t0 + base, L)] = acc
            return 0

        lax.fori_loop(0, C // L, grp, 0)

    @pl.when(ntot > 0)
    def _():
        start(0, buf0, sem0)

    def chunk_body(i, _):
        @pl.when(i % 2 == 0)
        def _():
            wait(buf0, sem0)

            @pl.when(i + 1 < ntot)
            def _():
                start(i + 1, buf1, sem1)

            compute_chunk(i, buf0)

        @pl.when(i % 2 == 1)
        def _():
            wait(buf1, sem1)

            @pl.when(i + 1 < ntot)
            def _():
                start(i + 1, buf0, sem0)

            compute_chunk(i, buf1)

        return 0

    lax.fori_loop(0, ntot, chunk_body, 0)

    for r, ln in ((0, ln0), (1, ln1)):
        def clean(g, _):
            tv = iota + g * L
            v = out_v[r, pl.ds(g * L, L)]
            out_v[r, pl.ds(g * L, L)] = jnp.where(tv < ln, v, 0)
            return 0

        lax.fori_loop(ln // L, T // L, clean, 0)

    pltpu.sync_copy(out_v, out_hbm.at[pl.ds(r0, ROWS_PER_W)])


@jax.jit
def kernel(unaries, lengths):
    mesh = plsc.VectorSubcoreMesh(core_axis_name="c", subcore_axis_name="s",
                                  num_cores=NC, num_subcores=NS)
    return pl.kernel(
        _sc_body,
        out_type=jax.ShapeDtypeStruct((B, T), jnp.int32),
        mesh=mesh,
        compiler_params=pltpu.CompilerParams(needs_layout_passes=False),
        scratch_types=[
            pltpu.VMEM((B,), jnp.int32),
            pltpu.VMEM((C, K), jnp.float32),
            pltpu.VMEM((C, K), jnp.float32),
            pltpu.VMEM((ROWS_PER_W, T), jnp.int32),
            pltpu.SemaphoreType.DMA,
            pltpu.SemaphoreType.DMA,
        ],
    )(unaries, lengths)
